# baseline (speedup 1.0000x reference)
import functools

import jax
import jax.numpy as jnp
from jax import lax
from jax.experimental import pallas as pl
from jax.experimental.pallas import tpu as pltpu
from jax.experimental.pallas import tpu_sc as plsc

B = 16384
D = 64
NT = 1000
NC = 2
NS = 16
NW = NC * NS
SCW = 2
CHW = SCW * 128
NKC = 7812
NSC = NKC // SCW
TAILBASE = NKC * 128
KSUB = 32
CAP = 768
LSZ = CAP + 16
SLOTS = 4096
OVC = 144
DUMP = B
OUTP = B + 128
PCH = 512

_mesh = plsc.VectorSubcoreMesh(core_axis_name="c", subcore_axis_name="s")


@functools.partial(
    pl.kernel,
    out_type=jax.ShapeDtypeStruct((OUTP,), jnp.float32),
    mesh=_mesh,
    compiler_params=pltpu.CompilerParams(
        needs_layout_passes=False, use_tc_tiling_on_sc=True),
    scratch_types=[
        pltpu.VMEM((D, NT), jnp.float32),
        pltpu.VMEM((D, CHW), jnp.float32),
        pltpu.VMEM((D, CHW), jnp.float32),
        pltpu.VMEM((4096,), jnp.float32),
        pltpu.VMEM((PCH,), jnp.int32),
        pltpu.VMEM((PCH,), jnp.int32),
        pltpu.VMEM((PCH,), jnp.int32),
        pltpu.VMEM((PCH,), jnp.int32),
        pltpu.VMEM((LSZ,), jnp.int32),
        pltpu.VMEM((LSZ,), jnp.int32),
        pltpu.VMEM((LSZ,), jnp.int32),
        pltpu.VMEM((SLOTS,), jnp.int32),
        pltpu.VMEM((SLOTS,), jnp.int32),
        pltpu.VMEM((SLOTS,), jnp.int32),
        pltpu.VMEM((SLOTS,), jnp.float32),
        pltpu.VMEM((SLOTS // 128, 128), jnp.int32),
        pltpu.VMEM((OVC,), jnp.int32),
        pltpu.VMEM((OVC,), jnp.int32),
        pltpu.VMEM((OVC,), jnp.int32),
        pltpu.VMEM((128,), jnp.float32),
        pltpu.VMEM((1, 128), jnp.int32),
        pltpu.SemaphoreType.DMA,
        pltpu.SemaphoreType.DMA,
        pltpu.SemaphoreType.DMA,
        pltpu.SemaphoreType.DMA,
    ],
)
def _sc_score(ent_hbm, type_hbm, embt_hbm, typet_hbm, tail_hbm, out_hbm,
              ttab, stA, stB, tl, eA, eB, tA, tB,
              le1, lb1, lt1, le2, lb2, lt2, scores, obidx,
              ove, ovb, ovt, ovs, ovi, semPA, semPB, semA, semB):
    wid = lax.axis_index("s") * NC + lax.axis_index("c")
    iota = lax.iota(jnp.int32, 16)

    g_lo = wid * 122 + jnp.minimum(wid, 2)
    nsub = jnp.where(wid < 2, 123, 122).astype(jnp.int32)
    lo_kc = g_lo * SCW
    hi_kc = (g_lo + nsub) * SCW + jnp.where(wid == NW - 1, 1, 0)

    pltpu.sync_copy(typet_hbm, ttab)
    pltpu.sync_copy(tail_hbm, tl)

    def xranks(m):
        mi = m.astype(jnp.int32)
        return plsc.cumsum(mi) - mi, plsc.all_reduce_population_count(m)

    def p1_issue(ch, ebuf, tbuf, sem):
        off = ch * PCH
        pltpu.async_copy(ent_hbm.at[pl.ds(off, PCH)], ebuf, sem)
        pltpu.async_copy(type_hbm.at[pl.ds(off, PCH)], tbuf, sem)

    def p1_drain(ebuf, tbuf, sem):
        pltpu.make_async_copy(ent_hbm.at[pl.ds(0, PCH)], ebuf, sem).wait()
        pltpu.make_async_copy(type_hbm.at[pl.ds(0, PCH)], tbuf, sem).wait()

    def p1_process(ch, ebuf, tbuf, carry):
        ptrv, optrv = carry
        for v in range(PCH // 16):
            p = 16 * v
            ev = ebuf[pl.ds(p, 16)]
            tv = tbuf[pl.ds(p, 16)]
            kcv = ev >> 7
            bv = ch * PCH + p + iota
            inr = (kcv >= lo_kc) & (kcv < hi_kc)
            mm = inr & (kcv < NKC)
            rank, pc = xranks(mm)
            dst = jnp.minimum(ptrv + rank, CAP)
            plsc.store_scatter(le1, [dst], ev, mask=mm)
            plsc.store_scatter(lb1, [dst], bv, mask=mm)
            plsc.store_scatter(lt1, [dst], tv, mask=mm)
            ptrv = ptrv + pc
            mo = inr & (kcv >= NKC)
            ro, po = xranks(mo)
            dsto = jnp.minimum(optrv + ro, OVC - 16)
            plsc.store_scatter(ove, [dsto], ev, mask=mo)
            plsc.store_scatter(ovb, [dsto], bv, mask=mo)
            plsc.store_scatter(ovt, [dsto], tv, mask=mo)
            optrv = optrv + po
        return ptrv, optrv

    zv = jnp.zeros((16,), jnp.int32)
    p1_issue(0, eA, tA, semPA)

    def p1_body(i, carry):
        ch = 2 * i
        p1_drain(eA, tA, semPA)
        p1_issue(ch + 1, eB, tB, semPB)
        carry = p1_process(ch, eA, tA, carry)
        p1_drain(eB, tB, semPB)

        @pl.when(ch + 2 < B // PCH)
        def _():
            p1_issue(ch + 2, eA, tA, semPA)

        return p1_process(ch + 1, eB, tB, carry)

    cntv, optrv = lax.fori_loop(0, B // PCH // 2, p1_body, (zv, zv))
    cnt = cntv[0]

    dumpv = jnp.full((16,), DUMP, jnp.int32)
    negv = jnp.full((16,), -1, jnp.int32)
    for v in range(SLOTS // 16):
        le2[pl.ds(16 * v, 16)] = negv
        lb2[pl.ds(16 * v, 16)] = dumpv
        lt2[pl.ds(16 * v, 16)] = zv

    def subchunk_of(ev):
        return ((ev >> 7) - lo_kc) // SCW

    def p2_body(sc, optrv):
        def inner(v, carry):
            slotv, optrv = carry
            p = 16 * v
            ev = le1[pl.ds(p, 16)]
            bv = lb1[pl.ds(p, 16)]
            tv = lt1[pl.ds(p, 16)]
            m = (subchunk_of(ev) == sc) & ((p + iota) < cnt)
            rank, pc = xranks(m)
            slot = slotv + rank
            mok = m & (slot < KSUB)
            dst = jnp.minimum(sc * KSUB + slot, SLOTS - 1)
            plsc.store_scatter(le2, [dst], ev, mask=mok)
            plsc.store_scatter(lb2, [dst], bv, mask=mok)
            plsc.store_scatter(lt2, [dst], tv, mask=mok)
            mo = m & (slot >= KSUB)
            ro, po = xranks(mo)
            dsto = jnp.minimum(optrv + ro, OVC - 16)
            plsc.store_scatter(ove, [dsto], ev, mask=mo)
            plsc.store_scatter(ovb, [dsto], bv, mask=mo)
            plsc.store_scatter(ovt, [dsto], tv, mask=mo)
            return slotv + pc, optrv + po

        _, optrv = lax.fori_loop(0, (cnt + 15) >> 4, inner, (zv, optrv))
        return optrv

    optrv = lax.fori_loop(0, nsub, p2_body, optrv)

    def p3_issue(sc, buf, sem):
        off = pl.multiple_of((g_lo + sc) * CHW, 128)
        for bt in range(8):
            pltpu.async_copy(
                embt_hbm.at[pl.ds(8 * bt, 8), pl.ds(off, CHW)],
                buf.at[pl.ds(8 * bt, 8)], sem)

    def p3_drain(buf, sem):
        for bt in range(8):
            pltpu.make_async_copy(
                embt_hbm.at[pl.ds(0, 8), pl.ds(0, CHW)],
                buf.at[pl.ds(8 * bt, 8)], sem).wait()

    def p3_process(sc, buf):
        kc0 = (g_lo + sc) * SCW
        for gofs in (0, 16):
            p = sc * KSUB + gofs
            ev = le2[pl.ds(p, 16)]
            tvr = jnp.clip(lt2[pl.ds(p, 16)], 0, NT - 1)
            colv = jnp.clip(((ev >> 7) - kc0) * 128 + (ev & 127), 0, CHW - 1)
            acc = jnp.zeros((16,), jnp.float32)
            for f in range(D):
                fc = jnp.full((16,), f, jnp.int32)
                em = plsc.load_gather(buf, [fc, colv])
                tt = plsc.load_gather(ttab, [fc, tvr])
                acc = acc + em * tt
            scores[pl.ds(p, 16)] = acc

    p3_issue(0, stA, semA)
    p3_issue(1, stB, semB)

    def p3_body(i, carry):
        sc0 = 2 * i
        p3_drain(stA, semA)
        p3_process(sc0, stA)

        @pl.when(sc0 + 2 < nsub)
        def _():
            p3_issue(sc0 + 2, stA, semA)

        p3_drain(stB, semB)
        p3_process(sc0 + 1, stB)

        @pl.when(sc0 + 3 < nsub)
        def _():
            p3_issue(sc0 + 3, stB, semB)

        return carry

    lax.fori_loop(0, nsub >> 1, p3_body, 0)

    @pl.when((nsub & 1) == 1)
    def _():
        p3_drain(stA, semA)
        p3_process(nsub - 1, stA)

    for r in range(SLOTS // 128):
        for k in range(8):
            obidx[r, pl.ds(16 * k, 16)] = lb2[pl.ds(r * 128 + 16 * k, 16)]
    copies = []
    for r in range(SLOTS // 128):
        copies.append(pltpu.async_copy(
            scores.at[pl.ds(r * 128, 128)], out_hbm.at[obidx.at[r]], semA))
    for cp in copies:
        cp.wait()

    novf = optrv[0]
    for k in range(8):
        ovi[0, pl.ds(16 * k, 16)] = dumpv

    def ovf_body(j, carry):
        ev16 = ove[pl.ds(j, 16)]
        bv16 = ovb[pl.ds(j, 16)]
        tv16 = ovt[pl.ds(j, 16)]
        e = ev16[0]
        b = bv16[0]
        t = jnp.clip(tv16[0], 0, NT - 1)
        kc = jnp.minimum(e >> 7, NKC - 1)

        @pl.when(e < TAILBASE)
        def _():
            off = pl.multiple_of(kc * 128, 128)
            pltpu.async_copy(
                embt_hbm.at[:, pl.ds(off, 128)],
                stA.at[:, pl.ds(0, 128)], semA).wait()

        acc = jnp.zeros((16,), jnp.float32)
        for c in range(D // 16):
            fv = iota + 16 * c
            tcv = jnp.full((16,), t, jnp.int32)
            tvals = plsc.load_gather(ttab, [fv, tcv])
            em = plsc.load_gather(
                stA, [fv, jnp.full((16,), e & 127, jnp.int32)])
            et = plsc.load_gather(tl, [fv * D + jnp.clip(e - TAILBASE, 0, 63)])
            evals = jnp.where(jnp.full((16,), e >= TAILBASE, jnp.bool_), et, em)
            acc = acc + evals * tvals
        s = jnp.sum(acc)
        ovs[pl.ds(0, 16)] = jnp.where(iota == 0, s, jnp.float32(0))
        ovi[0, pl.ds(0, 16)] = jnp.where(iota == 0, b, DUMP)
        pltpu.async_copy(ovs, out_hbm.at[ovi.at[0]], semA).wait()
        return carry

    lax.fori_loop(0, jnp.minimum(novf, OVC - 16), ovf_body, 0)


def kernel(ent, ent_type, batch_type, ent_emb, type_emb):
    del batch_type
    tail = ent_emb[TAILBASE:].T.reshape(-1)
    score = _sc_score(ent.astype(jnp.int32), ent_type.astype(jnp.int32),
                      ent_emb.T, type_emb.T, tail)
    return score[:B, None]


# speedup vs baseline: 30.1777x; 30.1777x over previous
import functools

import jax
import jax.numpy as jnp
from jax import lax
from jax.experimental import pallas as pl
from jax.experimental.pallas import tpu as pltpu
from jax.experimental.pallas import tpu_sc as plsc

B = 16384
D = 64
NT = 1000
NC = 2
NS = 16
NW = NC * NS
BPW = B // NW

_mesh = plsc.VectorSubcoreMesh(core_axis_name="c", subcore_axis_name="s")


@functools.partial(
    pl.kernel,
    out_type=jax.ShapeDtypeStruct((B,), jnp.float32),
    mesh=_mesh,
    compiler_params=pltpu.CompilerParams(
        needs_layout_passes=False, use_tc_tiling_on_sc=True),
    scratch_types=[
        pltpu.VMEM((D, NT), jnp.float32),
        pltpu.VMEM((D, 128), jnp.float32),
        pltpu.VMEM((D, 128), jnp.float32),
        pltpu.VMEM((BPW,), jnp.float32),
        pltpu.VMEM((4, 128), jnp.int32),
        pltpu.VMEM((BPW,), jnp.int32),
        pltpu.VMEM((BPW,), jnp.int32),
        pltpu.SemaphoreType.DMA,
        pltpu.SemaphoreType.DMA,
        pltpu.SemaphoreType.DMA,
    ],
)
def _sc_score(ent_hbm, type_hbm, embt_hbm, typet_hbm, out_hbm,
              ttab, tbufA, tbufB, outv, bidx, eidx_v, tidx_v,
              semA, semB, sem2):
    wid = lax.axis_index("s") * NC + lax.axis_index("c")
    base = wid * BPW

    pltpu.sync_copy(ent_hbm.at[pl.ds(base, BPW)], eidx_v)
    pltpu.sync_copy(type_hbm.at[pl.ds(base, BPW)], tidx_v)
    pltpu.sync_copy(typet_hbm, ttab)

    iota = lax.iota(jnp.int32, 16)
    for r in range(4):
        for k in range(8):
            bidx[r, pl.ds(16 * k, 16)] = base + r * 128 + 16 * k + iota

    def issue(e, buf, sem):
        off = pl.multiple_of((e // 128) * 128, 128)
        pltpu.async_copy(embt_hbm.at[:, pl.ds(off, 128)], buf, sem)

    def drain(buf, sem):
        pltpu.make_async_copy(
            embt_hbm.at[:, pl.ds(0, 128)], buf, sem).wait()

    issue(eidx_v[pl.ds(0, 16)][0], tbufA, semA)

    def group(g, carry):
        svec = jnp.zeros((16,), jnp.float32)
        ev16 = eidx_v[pl.ds(g * 16, 16)]
        gn = jnp.minimum(g + 1, BPW // 16 - 1)
        ev16n = eidx_v[pl.ds(gn * 16, 16)]
        tv16 = tidx_v[pl.ds(g * 16, 16)]
        for u in range(16):
            buf, sem = (tbufA, semA) if u % 2 == 0 else (tbufB, semB)
            nbuf, nsem = (tbufB, semB) if u % 2 == 0 else (tbufA, semA)
            drain(buf, sem)
            e_next = ev16[u + 1] if u < 15 else ev16n[0]
            issue(e_next, nbuf, nsem)
            e = ev16[u]
            ec = e & 127
            tc = tv16[u]
            acc = jnp.zeros((16,), jnp.float32)
            for c in range(D // 16):
                fv = iota + 16 * c
                ev = plsc.load_gather(buf, [fv, jnp.full((16,), ec, jnp.int32)])
                tv = plsc.load_gather(ttab, [fv, jnp.full((16,), tc, jnp.int32)])
                acc = acc + ev * tv
            svec = jnp.where(iota == u, jnp.sum(acc), svec)
        outv[pl.ds(g * 16, 16)] = svec
        return carry

    lax.fori_loop(0, BPW // 16, group, 0)
    drain(tbufA, semA)

    copies = []
    for r in range(4):
        copies.append(pltpu.async_copy(
            outv.at[pl.ds(r * 128, 128)], out_hbm.at[bidx.at[r]], sem2))
    for cp in copies:
        cp.wait()


def kernel(ent, ent_type, batch_type, ent_emb, type_emb):
    del batch_type
    score = _sc_score(ent.astype(jnp.int32), ent_type.astype(jnp.int32),
                      ent_emb.T, type_emb.T)
    return score[:, None]
